# Initial kernel scaffold; baseline (speedup 1.0000x reference)
import functools

import jax
import jax.numpy as jnp
from jax import lax
from jax.experimental import pallas as pl
from jax.experimental.pallas import tpu as pltpu
from jax.experimental.pallas import tpu_sc as plsc

_HIGHEST = lax.Precision.HIGHEST

_NUM_WORKERS = 32


def _router_body(tb, x_ref, wg_ref, pos_ref, meta_ref):
    t = x_ref.shape[0]
    e = wg_ref.shape[0]
    x = x_ref[...]
    w = wg_ref[...]
    logits = lax.dot_general(x, w, (((1,), (1,)), ((), ())), precision=_HIGHEST)
    m = jnp.max(logits, axis=1, keepdims=True)
    lane = lax.broadcasted_iota(jnp.int32, (t, e), 1)
    sel = jnp.min(jnp.where(logits == m, lane, e), axis=1, keepdims=True)
    one_hot = (lane == sel).astype(jnp.float32)

    ones = jnp.ones((t, 1), jnp.float32)
    counts = lax.dot_general(one_hot, ones, (((0,), (0,)), ((), ())),
                             precision=_HIGHEST)
    nblk = jnp.ceil(counts / tb)
    ei = lax.broadcasted_iota(jnp.int32, (e, e), 0)
    ej = lax.broadcasted_iota(jnp.int32, (e, e), 1)
    mle = (ej <= ei).astype(jnp.float32)
    cum = lax.dot_general(mle, nblk, (((1,), (0,)), ((), ())),
                          precision=_HIGHEST)
    poff = (cum - nblk) * tb

    ri = lax.broadcasted_iota(jnp.int32, (t, t), 0)
    ci = lax.broadcasted_iota(jnp.int32, (t, t), 1)
    lstrict = (ci < ri).astype(jnp.float32)
    ranks = lax.dot_general(lstrict, one_hot, (((1,), (0,)), ((), ())),
                            precision=_HIGHEST)
    rank_sel = jnp.sum(ranks * one_hot, axis=1, keepdims=True)
    off_sel = lax.dot_general(one_hot, poff, (((1,), (0,)), ((), ())),
                              precision=_HIGHEST)
    pos_ref[...] = (off_sel + rank_sel).astype(jnp.int32)

    total = jnp.max(cum)
    bl = lax.broadcasted_iota(jnp.float32, (1, meta_ref.shape[1]), 1)
    beff = jnp.minimum(bl, total - 1.0)
    bexp = jnp.sum((cum <= beff).astype(jnp.float32), axis=0, keepdims=True)
    bval = (bl < total).astype(jnp.float32)
    meta_ref[0:1, :] = bexp.astype(jnp.int32)
    meta_ref[1:2, :] = bval.astype(jnp.int32)


def _router(x, wg, tb):
    t, _ = x.shape
    return pl.pallas_call(
        functools.partial(_router_body, tb),
        out_shape=(
            jax.ShapeDtypeStruct((t, 1), jnp.int32),
            jax.ShapeDtypeStruct((2, 128), jnp.int32),
        ),
    )(x, wg)


def _sc_mesh():
    return plsc.VectorSubcoreMesh(core_axis_name="c", subcore_axis_name="s")


def _dispatch(x, pos, npad):
    t, h = x.shape
    ch = t // _NUM_WORKERS

    @functools.partial(
        pl.kernel,
        mesh=_sc_mesh(),
        out_type=jax.ShapeDtypeStruct((npad, h), jnp.float32),
        scratch_types=[
            pltpu.VMEM((ch,), jnp.int32),
            pltpu.VMEM((ch, h), jnp.float32),
            pltpu.SemaphoreType.DMA,
        ],
    )
    def k(x_hbm, idx_hbm, xs_hbm, idx_v, rows_v, sem):
        wid = lax.axis_index("s") * 2 + lax.axis_index("c")
        base = wid * ch
        pltpu.sync_copy(idx_hbm.at[pl.ds(base, ch)], idx_v)
        pltpu.sync_copy(x_hbm.at[pl.ds(base, ch)], rows_v)
        pltpu.async_copy(rows_v, xs_hbm.at[idx_v], sem).wait()

    return k(x, pos)


def _combine(ys, pos, t):
    h = ys.shape[1]
    ch = t // _NUM_WORKERS

    @functools.partial(
        pl.kernel,
        mesh=_sc_mesh(),
        out_type=jax.ShapeDtypeStruct((t, h), jnp.float32),
        scratch_types=[
            pltpu.VMEM((ch,), jnp.int32),
            pltpu.VMEM((ch, h), jnp.float32),
            pltpu.SemaphoreType.DMA,
        ],
    )
    def k(ys_hbm, idx_hbm, out_hbm, idx_v, rows_v, sem):
        wid = lax.axis_index("s") * 2 + lax.axis_index("c")
        base = wid * ch
        pltpu.sync_copy(idx_hbm.at[pl.ds(base, ch)], idx_v)
        pltpu.async_copy(ys_hbm.at[idx_v], rows_v, sem).wait()
        pltpu.sync_copy(rows_v, out_hbm.at[pl.ds(base, ch)])

    return k(ys, pos)


def _mlp_block(x, g_w, u_w, d_w):
    g = lax.dot_general(x, g_w, (((1,), (1,)), ((), ())),
                        preferred_element_type=jnp.float32)
    u = lax.dot_general(x, u_w, (((1,), (1,)), ((), ())),
                        preferred_element_type=jnp.float32)
    a = g * jax.nn.sigmoid(g) * u
    return lax.dot_general(a, d_w, (((1,), (1,)), ((), ())),
                           preferred_element_type=jnp.float32)


def _grouped_body(be_ref, bv_ref, xs_ref, wg_ref, wu_ref, wd_ref, ys_ref):
    b = pl.program_id(0)

    @pl.when(bv_ref[b] == 1)
    def _():
        ys_ref[...] = _mlp_block(xs_ref[...], wg_ref[0], wu_ref[0], wd_ref[0])


def _grouped(xs, wg_e, wu_e, wd_e, bexp, bval, tb, nb):
    npad, h = xs.shape
    f = wg_e.shape[1]
    grid_spec = pltpu.PrefetchScalarGridSpec(
        num_scalar_prefetch=2,
        grid=(nb,),
        in_specs=[
            pl.BlockSpec((tb, h), lambda b, be, bv: (b, 0)),
            pl.BlockSpec((1, f, h), lambda b, be, bv: (be[b], 0, 0)),
            pl.BlockSpec((1, f, h), lambda b, be, bv: (be[b], 0, 0)),
            pl.BlockSpec((1, h, f), lambda b, be, bv: (be[b], 0, 0)),
        ],
        out_specs=pl.BlockSpec((tb, h), lambda b, be, bv: (b, 0)),
    )
    return pl.pallas_call(
        _grouped_body,
        grid_spec=grid_spec,
        out_shape=jax.ShapeDtypeStruct((npad, h), jnp.float32),
    )(bexp, bval, xs, wg_e, wu_e, wd_e)


def _shared_body(x_ref, sg_ref, su_ref, sd_ref, o_ref):
    o_ref[...] = _mlp_block(x_ref[...], sg_ref[...], su_ref[...], sd_ref[...])


def _shared(x, sg, su, sd, tb):
    t, h = x.shape
    f = sg.shape[0]
    return pl.pallas_call(
        _shared_body,
        grid=(t // tb,),
        in_specs=[
            pl.BlockSpec((tb, h), lambda b: (b, 0)),
            pl.BlockSpec((f, h), lambda b: (0, 0)),
            pl.BlockSpec((f, h), lambda b: (0, 0)),
            pl.BlockSpec((h, f), lambda b: (0, 0)),
        ],
        out_specs=pl.BlockSpec((tb, h), lambda b: (b, 0)),
        out_shape=jax.ShapeDtypeStruct((t, h), jnp.float32),
    )(x, sg, su, sd)


def _add_body(a_ref, b_ref, o_ref):
    o_ref[...] = a_ref[...] + b_ref[...]


def _add(a, b, tb):
    t, h = a.shape
    return pl.pallas_call(
        _add_body,
        grid=(t // tb,),
        in_specs=[
            pl.BlockSpec((tb, h), lambda i: (i, 0)),
            pl.BlockSpec((tb, h), lambda i: (i, 0)),
        ],
        out_specs=pl.BlockSpec((tb, h), lambda i: (i, 0)),
        out_shape=jax.ShapeDtypeStruct((t, h), jnp.float32),
    )(a, b)


def kernel(hidden_states, wg, Wg, Wu, Wd, Sg, Su, Sd):
    bsz, s, h = hidden_states.shape
    t = bsz * s
    e = wg.shape[0]
    tb = 256
    nb = t // tb + e
    npad = nb * tb

    x = hidden_states.reshape(t, h)
    pos2, meta = _router(x, wg, tb)
    pos = pos2.reshape(t)
    bexp = meta[0, :nb]
    bval = meta[1, :nb]

    xs = _dispatch(x, pos, npad)
    ys = _grouped(xs, Wg, Wu, Wd, bexp, bval, tb, nb)
    sh = _shared(x, Sg, Su, Sd, tb)
    moe = _combine(ys, pos, t)
    out = _add(moe, sh, tb)
    return out.reshape(bsz, s, h)


# speedup vs baseline: 3.4519x; 3.4519x over previous
import functools

import jax
import jax.numpy as jnp
from jax import lax
from jax.experimental import pallas as pl
from jax.experimental.pallas import tpu as pltpu
from jax.experimental.pallas import tpu_sc as plsc

_HIGHEST = lax.Precision.HIGHEST

_NUM_WORKERS = 32


def _router_body(tb, x_ref, wg_ref, pos_ref, meta_ref):
    t = x_ref.shape[0]
    e = wg_ref.shape[0]
    x = x_ref[...]
    w = wg_ref[...]
    logits = lax.dot_general(x, w, (((1,), (1,)), ((), ())))
    m = jnp.max(logits, axis=1, keepdims=True)
    lane = lax.broadcasted_iota(jnp.int32, (t, e), 1)
    sel = jnp.min(jnp.where(logits == m, lane, e), axis=1, keepdims=True)
    one_hot = (lane == sel).astype(jnp.float32)

    ones = jnp.ones((t, 1), jnp.float32)
    counts = lax.dot_general(one_hot, ones, (((0,), (0,)), ((), ())),
                             precision=_HIGHEST)
    nblk = jnp.ceil(counts / tb)
    ei = lax.broadcasted_iota(jnp.int32, (e, e), 0)
    ej = lax.broadcasted_iota(jnp.int32, (e, e), 1)
    mle = (ej <= ei).astype(jnp.float32)
    cum = lax.dot_general(mle, nblk, (((1,), (0,)), ((), ())),
                          precision=_HIGHEST)
    poff = (cum - nblk) * tb

    ri = lax.broadcasted_iota(jnp.int32, (t, t), 0)
    ci = lax.broadcasted_iota(jnp.int32, (t, t), 1)
    lstrict = (ci < ri).astype(jnp.float32)
    ranks = lax.dot_general(lstrict, one_hot, (((1,), (0,)), ((), ())),
                            precision=_HIGHEST)
    rank_sel = jnp.sum(ranks * one_hot, axis=1, keepdims=True)
    off_sel = lax.dot_general(one_hot, poff, (((1,), (0,)), ((), ())),
                              precision=_HIGHEST)
    pos_ref[...] = (off_sel + rank_sel).astype(jnp.int32)

    total = jnp.max(cum)
    bl = lax.broadcasted_iota(jnp.int32, (1, meta_ref.shape[1]), 1).astype(
        jnp.float32)
    beff = jnp.minimum(bl, total - 1.0)
    bexp = jnp.sum((cum <= beff).astype(jnp.float32), axis=0, keepdims=True)
    bval = (bl < total).astype(jnp.float32)
    meta_ref[0:1, :] = bexp.astype(jnp.int32)
    meta_ref[1:2, :] = bval.astype(jnp.int32)


def _router(x, wg, tb):
    t, _ = x.shape
    return pl.pallas_call(
        functools.partial(_router_body, tb),
        out_shape=(
            jax.ShapeDtypeStruct((t, 1), jnp.int32),
            jax.ShapeDtypeStruct((2, 128), jnp.int32),
        ),
    )(x, wg)


def _sc_mesh():
    return plsc.VectorSubcoreMesh(core_axis_name="c", subcore_axis_name="s")


def _dispatch(x, pos, npad):
    t, h = x.shape
    ch = t // _NUM_WORKERS

    @functools.partial(
        pl.kernel,
        mesh=_sc_mesh(),
        out_type=jax.ShapeDtypeStruct((npad, h), jnp.float32),
        scratch_types=[
            pltpu.VMEM((ch,), jnp.int32),
            pltpu.VMEM((ch, h), jnp.float32),
            pltpu.SemaphoreType.DMA,
        ],
    )
    def k(x_hbm, idx_hbm, xs_hbm, idx_v, rows_v, sem):
        wid = lax.axis_index("s") * 2 + lax.axis_index("c")
        base = wid * ch
        pltpu.sync_copy(idx_hbm.at[pl.ds(base, ch)], idx_v)
        pltpu.sync_copy(x_hbm.at[pl.ds(base, ch)], rows_v)
        pltpu.async_copy(rows_v, xs_hbm.at[idx_v], sem).wait()

    return k(x, pos)


def _combine(ys, pos, t):
    h = ys.shape[1]
    ch = t // _NUM_WORKERS

    @functools.partial(
        pl.kernel,
        mesh=_sc_mesh(),
        out_type=jax.ShapeDtypeStruct((t, h), jnp.float32),
        scratch_types=[
            pltpu.VMEM((ch,), jnp.int32),
            pltpu.VMEM((ch, h), jnp.float32),
            pltpu.SemaphoreType.DMA,
        ],
    )
    def k(ys_hbm, idx_hbm, out_hbm, idx_v, rows_v, sem):
        wid = lax.axis_index("s") * 2 + lax.axis_index("c")
        base = wid * ch
        pltpu.sync_copy(idx_hbm.at[pl.ds(base, ch)], idx_v)
        pltpu.async_copy(ys_hbm.at[idx_v], rows_v, sem).wait()
        pltpu.sync_copy(rows_v, out_hbm.at[pl.ds(base, ch)])

    return k(ys, pos)


def _mlp_block(x, g_w, u_w, d_w):
    g = lax.dot_general(x, g_w, (((1,), (1,)), ((), ())),
                        preferred_element_type=jnp.float32)
    u = lax.dot_general(x, u_w, (((1,), (1,)), ((), ())),
                        preferred_element_type=jnp.float32)
    a = g * jax.nn.sigmoid(g) * u
    return lax.dot_general(a, d_w, (((1,), (1,)), ((), ())),
                           preferred_element_type=jnp.float32)


def _grouped_body(be_ref, bv_ref, xs_ref, wg_ref, wu_ref, wd_ref, ys_ref):
    b = pl.program_id(0)

    @pl.when(bv_ref[b] == 1)
    def _():
        ys_ref[...] = _mlp_block(xs_ref[...], wg_ref[0], wu_ref[0], wd_ref[0])


def _grouped(xs, wg_e, wu_e, wd_e, bexp, bval, tb, nb):
    npad, h = xs.shape
    f = wg_e.shape[1]
    grid_spec = pltpu.PrefetchScalarGridSpec(
        num_scalar_prefetch=2,
        grid=(nb,),
        in_specs=[
            pl.BlockSpec((tb, h), lambda b, be, bv: (b, 0)),
            pl.BlockSpec((1, f, h), lambda b, be, bv: (be[b], 0, 0)),
            pl.BlockSpec((1, f, h), lambda b, be, bv: (be[b], 0, 0)),
            pl.BlockSpec((1, h, f), lambda b, be, bv: (be[b], 0, 0)),
        ],
        out_specs=pl.BlockSpec((tb, h), lambda b, be, bv: (b, 0)),
    )
    return pl.pallas_call(
        _grouped_body,
        grid_spec=grid_spec,
        out_shape=jax.ShapeDtypeStruct((npad, h), jnp.float32),
    )(bexp, bval, xs, wg_e, wu_e, wd_e)


def _shared_body(x_ref, sg_ref, su_ref, sd_ref, o_ref):
    o_ref[...] = _mlp_block(x_ref[...], sg_ref[...], su_ref[...], sd_ref[...])


def _shared(x, sg, su, sd, tb):
    t, h = x.shape
    f = sg.shape[0]
    return pl.pallas_call(
        _shared_body,
        grid=(t // tb,),
        in_specs=[
            pl.BlockSpec((tb, h), lambda b: (b, 0)),
            pl.BlockSpec((f, h), lambda b: (0, 0)),
            pl.BlockSpec((f, h), lambda b: (0, 0)),
            pl.BlockSpec((h, f), lambda b: (0, 0)),
        ],
        out_specs=pl.BlockSpec((tb, h), lambda b: (b, 0)),
        out_shape=jax.ShapeDtypeStruct((t, h), jnp.float32),
    )(x, sg, su, sd)


def _add_body(a_ref, b_ref, o_ref):
    o_ref[...] = a_ref[...] + b_ref[...]


def _add(a, b, tb):
    t, h = a.shape
    return pl.pallas_call(
        _add_body,
        grid=(t // tb,),
        in_specs=[
            pl.BlockSpec((tb, h), lambda i: (i, 0)),
            pl.BlockSpec((tb, h), lambda i: (i, 0)),
        ],
        out_specs=pl.BlockSpec((tb, h), lambda i: (i, 0)),
        out_shape=jax.ShapeDtypeStruct((t, h), jnp.float32),
    )(a, b)


def kernel(hidden_states, wg, Wg, Wu, Wd, Sg, Su, Sd):
    bsz, s, h = hidden_states.shape
    t = bsz * s
    e = wg.shape[0]
    tb = 256
    nb = t // tb + e
    npad = nb * tb

    x = hidden_states.reshape(t, h)
    pos2, meta = _router(x, wg, tb)
    pos = pos2.reshape(t)
    bexp = meta[0, :nb]
    bval = meta[1, :nb]

    xs = _dispatch(x, pos, npad)
    ys = _grouped(xs, Wg, Wu, Wd, bexp, bval, tb, nb)
    sh = _shared(x, Sg, Su, Sd, tb)
    moe = _combine(ys, pos, t)
    out = _add(moe, sh, tb)
    return out.reshape(bsz, s, h)


# speedup vs baseline: 3.6732x; 1.0641x over previous
import functools

import jax
import jax.numpy as jnp
from jax import lax
from jax.experimental import pallas as pl
from jax.experimental.pallas import tpu as pltpu
from jax.experimental.pallas import tpu_sc as plsc

_HIGHEST = lax.Precision.HIGHEST

_NUM_WORKERS = 32


def _router_body(tb, x_ref, wg_ref, pos_ref, meta_ref, oh_ref):
    t = x_ref.shape[0]
    e = wg_ref.shape[0]
    x = x_ref[...]
    w = wg_ref[...]
    logits = lax.dot_general(x, w, (((1,), (1,)), ((), ())))
    m = jnp.max(logits, axis=1, keepdims=True)
    lane = lax.broadcasted_iota(jnp.int32, (t, e), 1)
    sel = jnp.min(jnp.where(logits == m, lane, e), axis=1, keepdims=True)
    one_hot = (lane == sel).astype(jnp.float32)

    ones = jnp.ones((t, 1), jnp.float32)
    counts = lax.dot_general(one_hot, ones, (((0,), (0,)), ((), ())),
                             precision=_HIGHEST)
    nblk = jnp.ceil(counts / tb)
    ei = lax.broadcasted_iota(jnp.int32, (e, e), 0)
    ej = lax.broadcasted_iota(jnp.int32, (e, e), 1)
    mle = (ej <= ei).astype(jnp.float32)
    cum = lax.dot_general(mle, nblk, (((1,), (0,)), ((), ())),
                          precision=_HIGHEST)
    poff = (cum - nblk) * tb

    oh_ref[...] = one_hot
    lsz = 256
    ri = lax.broadcasted_iota(jnp.int32, (lsz, lsz), 0)
    ci = lax.broadcasted_iota(jnp.int32, (lsz, lsz), 1)
    lstrict = (ci < ri).astype(jnp.float32)

    def chunk(i, carry):
        oh = oh_ref[pl.ds(i * lsz, lsz), :]
        ranks = lax.dot_general(lstrict, oh, (((1,), (0,)), ((), ())),
                                precision=_HIGHEST) + carry
        rank_sel = jnp.sum(ranks * oh, axis=1, keepdims=True)
        off_sel = lax.dot_general(oh, poff, (((1,), (0,)), ((), ())),
                                  precision=_HIGHEST)
        pos_ref[pl.ds(i * lsz, lsz), :] = (off_sel + rank_sel).astype(
            jnp.int32)
        return carry + jnp.sum(oh, axis=0, keepdims=True)

    lax.fori_loop(0, t // lsz, chunk, jnp.zeros((1, e), jnp.float32))

    total = jnp.max(cum)
    bl = lax.broadcasted_iota(jnp.int32, (1, meta_ref.shape[1]), 1).astype(
        jnp.float32)
    beff = jnp.minimum(bl, total - 1.0)
    bexp = jnp.sum((cum <= beff).astype(jnp.float32), axis=0, keepdims=True)
    bval = (bl < total).astype(jnp.float32)
    meta_ref[0:1, :] = bexp.astype(jnp.int32)
    meta_ref[1:2, :] = bval.astype(jnp.int32)


def _router(x, wg, tb):
    t, _ = x.shape
    e = wg.shape[0]
    return pl.pallas_call(
        functools.partial(_router_body, tb),
        out_shape=(
            jax.ShapeDtypeStruct((t, 1), jnp.int32),
            jax.ShapeDtypeStruct((2, 128), jnp.int32),
        ),
        scratch_shapes=[pltpu.VMEM((t, e), jnp.float32)],
    )(x, wg)


def _sc_mesh():
    return plsc.VectorSubcoreMesh(core_axis_name="c", subcore_axis_name="s")


def _dispatch(x, pos, npad):
    t, h = x.shape
    ch = t // _NUM_WORKERS

    @functools.partial(
        pl.kernel,
        mesh=_sc_mesh(),
        out_type=jax.ShapeDtypeStruct((npad, h), jnp.float32),
        scratch_types=[
            pltpu.VMEM((ch,), jnp.int32),
            pltpu.VMEM((ch, h), jnp.float32),
            pltpu.SemaphoreType.DMA,
        ],
    )
    def k(x_hbm, idx_hbm, xs_hbm, idx_v, rows_v, sem):
        wid = lax.axis_index("s") * 2 + lax.axis_index("c")
        base = wid * ch
        pltpu.sync_copy(idx_hbm.at[pl.ds(base, ch)], idx_v)
        pltpu.sync_copy(x_hbm.at[pl.ds(base, ch)], rows_v)
        pltpu.async_copy(rows_v, xs_hbm.at[idx_v], sem).wait()

    return k(x, pos)


def _combine(ys, pos, t):
    h = ys.shape[1]
    ch = t // _NUM_WORKERS

    @functools.partial(
        pl.kernel,
        mesh=_sc_mesh(),
        out_type=jax.ShapeDtypeStruct((t, h), jnp.float32),
        scratch_types=[
            pltpu.VMEM((ch,), jnp.int32),
            pltpu.VMEM((ch, h), jnp.float32),
            pltpu.SemaphoreType.DMA,
        ],
    )
    def k(ys_hbm, idx_hbm, out_hbm, idx_v, rows_v, sem):
        wid = lax.axis_index("s") * 2 + lax.axis_index("c")
        base = wid * ch
        pltpu.sync_copy(idx_hbm.at[pl.ds(base, ch)], idx_v)
        pltpu.async_copy(ys_hbm.at[idx_v], rows_v, sem).wait()
        pltpu.sync_copy(rows_v, out_hbm.at[pl.ds(base, ch)])

    return k(ys, pos)


def _mlp_block(x, g_w, u_w, d_w):
    g = lax.dot_general(x, g_w, (((1,), (1,)), ((), ())),
                        preferred_element_type=jnp.float32)
    u = lax.dot_general(x, u_w, (((1,), (1,)), ((), ())),
                        preferred_element_type=jnp.float32)
    a = g * jax.nn.sigmoid(g) * u
    return lax.dot_general(a, d_w, (((1,), (1,)), ((), ())),
                           preferred_element_type=jnp.float32)


def _grouped_body(be_ref, bv_ref, xs_ref, wg_ref, wu_ref, wd_ref, ys_ref):
    b = pl.program_id(0)

    @pl.when(bv_ref[b] == 1)
    def _():
        ys_ref[...] = _mlp_block(xs_ref[...], wg_ref[0], wu_ref[0], wd_ref[0])


def _grouped(xs, wg_e, wu_e, wd_e, bexp, bval, tb, nb):
    npad, h = xs.shape
    f = wg_e.shape[1]
    grid_spec = pltpu.PrefetchScalarGridSpec(
        num_scalar_prefetch=2,
        grid=(nb,),
        in_specs=[
            pl.BlockSpec((tb, h), lambda b, be, bv: (b, 0)),
            pl.BlockSpec((1, f, h), lambda b, be, bv: (be[b], 0, 0)),
            pl.BlockSpec((1, f, h), lambda b, be, bv: (be[b], 0, 0)),
            pl.BlockSpec((1, h, f), lambda b, be, bv: (be[b], 0, 0)),
        ],
        out_specs=pl.BlockSpec((tb, h), lambda b, be, bv: (b, 0)),
    )
    return pl.pallas_call(
        _grouped_body,
        grid_spec=grid_spec,
        out_shape=jax.ShapeDtypeStruct((npad, h), jnp.float32),
        compiler_params=pltpu.CompilerParams(
            dimension_semantics=("parallel",)),
    )(bexp, bval, xs, wg_e, wu_e, wd_e)


def _shared_body(x_ref, sg_ref, su_ref, sd_ref, o_ref):
    o_ref[...] = _mlp_block(x_ref[...], sg_ref[...], su_ref[...], sd_ref[...])


def _shared(x, sg, su, sd, tb):
    t, h = x.shape
    f = sg.shape[0]
    return pl.pallas_call(
        _shared_body,
        grid=(t // tb,),
        in_specs=[
            pl.BlockSpec((tb, h), lambda b: (b, 0)),
            pl.BlockSpec((f, h), lambda b: (0, 0)),
            pl.BlockSpec((f, h), lambda b: (0, 0)),
            pl.BlockSpec((h, f), lambda b: (0, 0)),
        ],
        out_specs=pl.BlockSpec((tb, h), lambda b: (b, 0)),
        out_shape=jax.ShapeDtypeStruct((t, h), jnp.float32),
        compiler_params=pltpu.CompilerParams(
            dimension_semantics=("parallel",)),
    )(x, sg, su, sd)


def _add_body(a_ref, b_ref, o_ref):
    o_ref[...] = a_ref[...] + b_ref[...]


def _add(a, b, tb):
    t, h = a.shape
    return pl.pallas_call(
        _add_body,
        grid=(t // tb,),
        in_specs=[
            pl.BlockSpec((tb, h), lambda i: (i, 0)),
            pl.BlockSpec((tb, h), lambda i: (i, 0)),
        ],
        out_specs=pl.BlockSpec((tb, h), lambda i: (i, 0)),
        out_shape=jax.ShapeDtypeStruct((t, h), jnp.float32),
        compiler_params=pltpu.CompilerParams(
            dimension_semantics=("parallel",)),
    )(a, b)


def kernel(hidden_states, wg, Wg, Wu, Wd, Sg, Su, Sd):
    bsz, s, h = hidden_states.shape
    t = bsz * s
    e = wg.shape[0]
    tb = 256
    nb = t // tb + e
    npad = nb * tb

    x = hidden_states.reshape(t, h)
    pos2, meta = _router(x, wg, tb)
    pos = pos2.reshape(t)
    bexp = meta[0, :nb]
    bval = meta[1, :nb]

    xs = _dispatch(x, pos, npad)
    ys = _grouped(xs, Wg, Wu, Wd, bexp, bval, tb, nb)
    sh = _shared(x, Sg, Su, Sd, tb)
    moe = _combine(ys, pos, t)
    out = _add(moe, sh, tb)
    return out.reshape(bsz, s, h)


# speedup vs baseline: 3.7767x; 1.0282x over previous
import functools

import jax
import jax.numpy as jnp
from jax import lax
from jax.experimental import pallas as pl
from jax.experimental.pallas import tpu as pltpu
from jax.experimental.pallas import tpu_sc as plsc

_HIGHEST = lax.Precision.HIGHEST

_NUM_WORKERS = 32


def _router_body(tb, x_ref, wg_ref, pos_ref, meta_ref, oh_ref):
    t = x_ref.shape[0]
    e = wg_ref.shape[0]
    x = x_ref[...]
    w = wg_ref[...]
    logits = lax.dot_general(x, w, (((1,), (1,)), ((), ())))
    m = jnp.max(logits, axis=1, keepdims=True)
    lane = lax.broadcasted_iota(jnp.int32, (t, e), 1)
    sel = jnp.min(jnp.where(logits == m, lane, e), axis=1, keepdims=True)
    one_hot = (lane == sel).astype(jnp.float32)

    ones = jnp.ones((t, 1), jnp.float32)
    counts = lax.dot_general(one_hot, ones, (((0,), (0,)), ((), ())),
                             precision=_HIGHEST)
    nblk = jnp.ceil(counts / tb)
    ei = lax.broadcasted_iota(jnp.int32, (e, e), 0)
    ej = lax.broadcasted_iota(jnp.int32, (e, e), 1)
    mle = (ej <= ei).astype(jnp.float32)
    cum = lax.dot_general(mle, nblk, (((1,), (0,)), ((), ())),
                          precision=_HIGHEST)
    poff = (cum - nblk) * tb

    oh_ref[...] = one_hot
    lsz = 256
    ri = lax.broadcasted_iota(jnp.int32, (lsz, lsz), 0)
    ci = lax.broadcasted_iota(jnp.int32, (lsz, lsz), 1)
    lstrict = (ci < ri).astype(jnp.float32)

    def chunk(i, carry):
        oh = oh_ref[pl.ds(i * lsz, lsz), :]
        ranks = lax.dot_general(lstrict, oh, (((1,), (0,)), ((), ())),
                                precision=_HIGHEST) + carry
        rank_sel = jnp.sum(ranks * oh, axis=1, keepdims=True)
        off_sel = lax.dot_general(oh, poff, (((1,), (0,)), ((), ())),
                                  precision=_HIGHEST)
        pos_ref[pl.ds(i * lsz, lsz), :] = (off_sel + rank_sel).astype(
            jnp.int32)
        return carry + jnp.sum(oh, axis=0, keepdims=True)

    lax.fori_loop(0, t // lsz, chunk, jnp.zeros((1, e), jnp.float32))

    total = jnp.max(cum)
    bl = lax.broadcasted_iota(jnp.int32, (1, meta_ref.shape[1]), 1).astype(
        jnp.float32)
    beff = jnp.minimum(bl, total - 1.0)
    bexp = jnp.sum((cum <= beff).astype(jnp.float32), axis=0, keepdims=True)
    bval = (bl < total).astype(jnp.float32)
    meta_ref[0:1, :] = bexp.astype(jnp.int32)
    meta_ref[1:2, :] = bval.astype(jnp.int32)


def _router(x, wg, tb):
    t, _ = x.shape
    e = wg.shape[0]
    return pl.pallas_call(
        functools.partial(_router_body, tb),
        out_shape=(
            jax.ShapeDtypeStruct((t, 1), jnp.int32),
            jax.ShapeDtypeStruct((2, 128), jnp.int32),
        ),
        scratch_shapes=[pltpu.VMEM((t, e), jnp.float32)],
    )(x, wg)


def _sc_mesh():
    return plsc.VectorSubcoreMesh(core_axis_name="c", subcore_axis_name="s")


def _dispatch(x, pos, npad):
    t, h = x.shape
    ch = t // _NUM_WORKERS

    @functools.partial(
        pl.kernel,
        mesh=_sc_mesh(),
        out_type=jax.ShapeDtypeStruct((npad, h), jnp.float32),
        scratch_types=[
            pltpu.VMEM((ch,), jnp.int32),
            pltpu.VMEM((ch, h), jnp.float32),
            pltpu.SemaphoreType.DMA,
        ],
    )
    def k(x_hbm, idx_hbm, xs_hbm, idx_v, rows_v, sem):
        wid = lax.axis_index("s") * 2 + lax.axis_index("c")
        base = wid * ch
        pltpu.sync_copy(idx_hbm.at[pl.ds(base, ch)], idx_v)
        pltpu.sync_copy(x_hbm.at[pl.ds(base, ch)], rows_v)
        pltpu.async_copy(rows_v, xs_hbm.at[idx_v], sem).wait()

    return k(x, pos)


def _combine(ys, pos, t):
    h = ys.shape[1]
    ch = t // _NUM_WORKERS

    @functools.partial(
        pl.kernel,
        mesh=_sc_mesh(),
        out_type=jax.ShapeDtypeStruct((t, h), jnp.float32),
        scratch_types=[
            pltpu.VMEM((ch,), jnp.int32),
            pltpu.VMEM((ch, h), jnp.float32),
            pltpu.SemaphoreType.DMA,
        ],
    )
    def k(ys_hbm, idx_hbm, out_hbm, idx_v, rows_v, sem):
        wid = lax.axis_index("s") * 2 + lax.axis_index("c")
        base = wid * ch
        pltpu.sync_copy(idx_hbm.at[pl.ds(base, ch)], idx_v)
        pltpu.async_copy(ys_hbm.at[idx_v], rows_v, sem).wait()
        pltpu.sync_copy(rows_v, out_hbm.at[pl.ds(base, ch)])

    return k(ys, pos)


def _mlp_block(x, g_w, u_w, d_w):
    g = lax.dot_general(x, g_w, (((1,), (1,)), ((), ())),
                        preferred_element_type=jnp.float32)
    u = lax.dot_general(x, u_w, (((1,), (1,)), ((), ())),
                        preferred_element_type=jnp.float32)
    a = g * jax.nn.sigmoid(g) * u
    return lax.dot_general(a, d_w, (((1,), (1,)), ((), ())),
                           preferred_element_type=jnp.float32)


def _grouped_body(be_ref, bv_ref, xs_ref, wg_ref, wu_ref, wd_ref, ys_ref):
    b = pl.program_id(0)

    @pl.when(bv_ref[b] == 1)
    def _():
        ys_ref[...] = _mlp_block(xs_ref[...], wg_ref[0], wu_ref[0], wd_ref[0])


def _grouped(xs, wg_e, wu_e, wd_e, bexp, bval, tb, nb):
    npad, h = xs.shape
    f = wg_e.shape[1]
    grid_spec = pltpu.PrefetchScalarGridSpec(
        num_scalar_prefetch=2,
        grid=(nb,),
        in_specs=[
            pl.BlockSpec((tb, h), lambda b, be, bv: (b, 0)),
            pl.BlockSpec((1, f, h), lambda b, be, bv: (be[b], 0, 0)),
            pl.BlockSpec((1, f, h), lambda b, be, bv: (be[b], 0, 0)),
            pl.BlockSpec((1, h, f), lambda b, be, bv: (be[b], 0, 0)),
        ],
        out_specs=pl.BlockSpec((tb, h), lambda b, be, bv: (b, 0)),
    )
    return pl.pallas_call(
        _grouped_body,
        grid_spec=grid_spec,
        out_shape=jax.ShapeDtypeStruct((npad, h), jnp.float32),
        compiler_params=pltpu.CompilerParams(
            dimension_semantics=("parallel",)),
    )(bexp, bval, xs, wg_e, wu_e, wd_e)


def _shared_body(x_ref, sg_ref, su_ref, sd_ref, moe_ref, o_ref):
    o_ref[...] = moe_ref[...] + _mlp_block(
        x_ref[...], sg_ref[...], su_ref[...], sd_ref[...])


def _shared_add(x, sg, su, sd, moe, tb):
    t, h = x.shape
    f = sg.shape[0]
    return pl.pallas_call(
        _shared_body,
        grid=(t // tb,),
        in_specs=[
            pl.BlockSpec((tb, h), lambda b: (b, 0)),
            pl.BlockSpec((f, h), lambda b: (0, 0)),
            pl.BlockSpec((f, h), lambda b: (0, 0)),
            pl.BlockSpec((h, f), lambda b: (0, 0)),
            pl.BlockSpec((tb, h), lambda b: (b, 0)),
        ],
        out_specs=pl.BlockSpec((tb, h), lambda b: (b, 0)),
        out_shape=jax.ShapeDtypeStruct((t, h), jnp.float32),
        compiler_params=pltpu.CompilerParams(
            dimension_semantics=("parallel",)),
    )(x, sg, su, sd, moe)


def kernel(hidden_states, wg, Wg, Wu, Wd, Sg, Su, Sd):
    bsz, s, h = hidden_states.shape
    t = bsz * s
    e = wg.shape[0]
    tb = 256
    nb = t // tb + e
    npad = nb * tb

    x = hidden_states.reshape(t, h)
    pos2, meta = _router(x, wg, tb)
    pos = pos2.reshape(t)
    bexp = meta[0, :nb]
    bval = meta[1, :nb]

    xs = _dispatch(x, pos, npad)
    ys = _grouped(xs, Wg, Wu, Wd, bexp, bval, tb, nb)
    moe = _combine(ys, pos, t)
    out = _shared_add(x, Sg, Su, Sd, moe, tb)
    return out.reshape(bsz, s, h)


# speedup vs baseline: 3.9044x; 1.0338x over previous
import functools

import jax
import jax.numpy as jnp
from jax import lax
from jax.experimental import pallas as pl
from jax.experimental.pallas import tpu as pltpu
from jax.experimental.pallas import tpu_sc as plsc

_HIGHEST = lax.Precision.HIGHEST

_NUM_WORKERS = 32


def _router_body(tb, x_ref, wg_ref, pos_ref, meta_ref, oh_ref):
    t = x_ref.shape[0]
    e = wg_ref.shape[0]
    x = x_ref[...]
    w = wg_ref[...]
    logits = lax.dot_general(x, w, (((1,), (1,)), ((), ())))
    m = jnp.max(logits, axis=1, keepdims=True)
    lane = lax.broadcasted_iota(jnp.int32, (t, e), 1)
    sel = jnp.min(jnp.where(logits == m, lane, e), axis=1, keepdims=True)
    one_hot = (lane == sel).astype(jnp.float32)

    ones = jnp.ones((t, 1), jnp.float32)
    counts = lax.dot_general(one_hot, ones, (((0,), (0,)), ((), ())),
                             precision=_HIGHEST)
    nblk = jnp.ceil(counts / tb)
    ei = lax.broadcasted_iota(jnp.int32, (e, e), 0)
    ej = lax.broadcasted_iota(jnp.int32, (e, e), 1)
    mle = (ej <= ei).astype(jnp.float32)
    cum = lax.dot_general(mle, nblk, (((1,), (0,)), ((), ())),
                          precision=_HIGHEST)
    poff = (cum - nblk) * tb

    oh_ref[...] = one_hot
    lsz = 256
    ri = lax.broadcasted_iota(jnp.int32, (lsz, lsz), 0)
    ci = lax.broadcasted_iota(jnp.int32, (lsz, lsz), 1)
    lstrict = (ci < ri).astype(jnp.float32)

    def chunk(i, carry):
        oh = oh_ref[pl.ds(i * lsz, lsz), :]
        ranks = lax.dot_general(lstrict, oh, (((1,), (0,)), ((), ())),
                                precision=_HIGHEST) + carry
        rank_sel = jnp.sum(ranks * oh, axis=1, keepdims=True)
        off_sel = lax.dot_general(oh, poff, (((1,), (0,)), ((), ())),
                                  precision=_HIGHEST)
        pos_ref[pl.ds(i * lsz, lsz), :] = (off_sel + rank_sel).astype(
            jnp.int32)
        return carry + jnp.sum(oh, axis=0, keepdims=True)

    lax.fori_loop(0, t // lsz, chunk, jnp.zeros((1, e), jnp.float32))

    total = jnp.max(cum)
    bl = lax.broadcasted_iota(jnp.int32, (1, meta_ref.shape[1]), 1).astype(
        jnp.float32)
    beff = jnp.minimum(bl, total - 1.0)
    bexp = jnp.sum((cum <= beff).astype(jnp.float32), axis=0, keepdims=True)
    bval = (bl < total).astype(jnp.float32)
    meta_ref[0:1, :] = bexp.astype(jnp.int32)
    meta_ref[1:2, :] = bval.astype(jnp.int32)
    meta_ref[2:3, :] = beff.astype(jnp.int32)


def _router(x, wg, tb):
    t, _ = x.shape
    e = wg.shape[0]
    return pl.pallas_call(
        functools.partial(_router_body, tb),
        out_shape=(
            jax.ShapeDtypeStruct((t, 1), jnp.int32),
            jax.ShapeDtypeStruct((3, 128), jnp.int32),
        ),
        scratch_shapes=[pltpu.VMEM((t, e), jnp.float32)],
    )(x, wg)


def _sc_mesh():
    return plsc.VectorSubcoreMesh(core_axis_name="c", subcore_axis_name="s")


def _dispatch(x, pos, npad):
    t, h = x.shape
    ch = t // _NUM_WORKERS

    @functools.partial(
        pl.kernel,
        mesh=_sc_mesh(),
        out_type=jax.ShapeDtypeStruct((npad, h), jnp.float32),
        scratch_types=[
            pltpu.VMEM((ch,), jnp.int32),
            pltpu.VMEM((ch, h), jnp.float32),
            pltpu.SemaphoreType.DMA,
        ],
    )
    def k(x_hbm, idx_hbm, xs_hbm, idx_v, rows_v, sem):
        wid = lax.axis_index("s") * 2 + lax.axis_index("c")
        base = wid * ch
        pltpu.sync_copy(idx_hbm.at[pl.ds(base, ch)], idx_v)
        pltpu.sync_copy(x_hbm.at[pl.ds(base, ch)], rows_v)
        pltpu.async_copy(rows_v, xs_hbm.at[idx_v], sem).wait()

    return k(x, pos)


def _combine(ys, pos, t):
    h = ys.shape[1]
    ch = t // _NUM_WORKERS

    @functools.partial(
        pl.kernel,
        mesh=_sc_mesh(),
        out_type=jax.ShapeDtypeStruct((t, h), jnp.float32),
        scratch_types=[
            pltpu.VMEM((ch,), jnp.int32),
            pltpu.VMEM((ch, h), jnp.float32),
            pltpu.SemaphoreType.DMA,
        ],
    )
    def k(ys_hbm, idx_hbm, out_hbm, idx_v, rows_v, sem):
        wid = lax.axis_index("s") * 2 + lax.axis_index("c")
        base = wid * ch
        pltpu.sync_copy(idx_hbm.at[pl.ds(base, ch)], idx_v)
        pltpu.async_copy(ys_hbm.at[idx_v], rows_v, sem).wait()
        pltpu.sync_copy(rows_v, out_hbm.at[pl.ds(base, ch)])

    return k(ys, pos)


def _mlp_block(x, g_w, u_w, d_w):
    g = lax.dot_general(x, g_w, (((1,), (1,)), ((), ())),
                        preferred_element_type=jnp.float32)
    u = lax.dot_general(x, u_w, (((1,), (1,)), ((), ())),
                        preferred_element_type=jnp.float32)
    a = g * jax.nn.sigmoid(g) * u
    return lax.dot_general(a, d_w, (((1,), (1,)), ((), ())),
                           preferred_element_type=jnp.float32)


def _grouped_body(be_ref, bv_ref, bx_ref, xs_ref, wg_ref, wu_ref, wd_ref,
                  ys_ref):
    b = pl.program_id(0)

    @pl.when(bv_ref[b] == 1)
    def _():
        ys_ref[...] = _mlp_block(xs_ref[...], wg_ref[0], wu_ref[0], wd_ref[0])


def _grouped(xs, wg_e, wu_e, wd_e, bexp, bval, bxsi, tb, nb):
    npad, h = xs.shape
    f = wg_e.shape[1]
    grid_spec = pltpu.PrefetchScalarGridSpec(
        num_scalar_prefetch=3,
        grid=(nb,),
        in_specs=[
            pl.BlockSpec((tb, h), lambda b, be, bv, bx: (bx[b], 0)),
            pl.BlockSpec((1, f, h), lambda b, be, bv, bx: (be[b], 0, 0)),
            pl.BlockSpec((1, f, h), lambda b, be, bv, bx: (be[b], 0, 0)),
            pl.BlockSpec((1, h, f), lambda b, be, bv, bx: (be[b], 0, 0)),
        ],
        out_specs=pl.BlockSpec(
            (tb, h),
            lambda b, be, bv, bx: (jnp.where(bv[b] == 1, b, nb), 0)),
    )
    return pl.pallas_call(
        _grouped_body,
        grid_spec=grid_spec,
        out_shape=jax.ShapeDtypeStruct((npad + tb, h), jnp.float32),
        compiler_params=pltpu.CompilerParams(
            dimension_semantics=("parallel",)),
    )(bexp, bval, bxsi, xs, wg_e, wu_e, wd_e)


def _shared_body(x_ref, sg_ref, su_ref, sd_ref, moe_ref, o_ref):
    o_ref[...] = moe_ref[...] + _mlp_block(
        x_ref[...], sg_ref[...], su_ref[...], sd_ref[...])


def _shared_add(x, sg, su, sd, moe, tb):
    t, h = x.shape
    f = sg.shape[0]
    return pl.pallas_call(
        _shared_body,
        grid=(t // tb,),
        in_specs=[
            pl.BlockSpec((tb, h), lambda b: (b, 0)),
            pl.BlockSpec((f, h), lambda b: (0, 0)),
            pl.BlockSpec((f, h), lambda b: (0, 0)),
            pl.BlockSpec((h, f), lambda b: (0, 0)),
            pl.BlockSpec((tb, h), lambda b: (b, 0)),
        ],
        out_specs=pl.BlockSpec((tb, h), lambda b: (b, 0)),
        out_shape=jax.ShapeDtypeStruct((t, h), jnp.float32),
        compiler_params=pltpu.CompilerParams(
            dimension_semantics=("parallel",)),
    )(x, sg, su, sd, moe)


def kernel(hidden_states, wg, Wg, Wu, Wd, Sg, Su, Sd):
    bsz, s, h = hidden_states.shape
    t = bsz * s
    e = wg.shape[0]
    tb = 256
    nb = t // tb + e
    npad = nb * tb

    x = hidden_states.reshape(t, h)
    pos2, meta = _router(x, wg, tb)
    pos = pos2.reshape(t)
    bexp = meta[0, :nb]
    bval = meta[1, :nb]
    bxsi = meta[2, :nb]

    xs = _dispatch(x, pos, npad)
    ys = _grouped(xs, Wg, Wu, Wd, bexp, bval, bxsi, tb, nb)
    moe = _combine(ys, pos, t)
    out = _shared_add(x, Sg, Su, Sd, moe, tb)
    return out.reshape(bsz, s, h)
